# Initial kernel scaffold; baseline (speedup 1.0000x reference)
import jax
import jax.numpy as jnp
from jax.experimental import pallas as pl


def kernel(x, edge_index, W0, b0, g0, be0, W1, b1, g1, be1, W2, b2, g2, be2, W3, b3, g3, be3, W4, b4, g4, be4, fcW, fcb):
    raise NotImplementedError("write your pallas kernel here")


# speedup vs baseline: 10.6179x; 10.6179x over previous
import functools

import jax
import jax.numpy as jnp
from jax import lax
from jax.experimental import pallas as pl
from jax.experimental.pallas import tpu as pltpu
from jax.experimental.pallas import tpu_sc as plsc

N = 10000
NP = 10240
E = 320000
NC = 2
NS = 16
NW = NC * NS
K = 125
CH = E // (NW * K)
RP = NP // NS
RB = 1000
NB = N // RB

_MESH = plsc.VectorSubcoreMesh(core_axis_name="c", subcore_axis_name="s")
_SC_PARAMS = pltpu.CompilerParams(use_tc_tiling_on_sc=False)


@functools.partial(jax.jit, static_argnums=(4,))
def _spmm(y, src, dst, zeros, F):
    @functools.partial(
        pl.kernel,
        out_type=jax.ShapeDtypeStruct((NC, NP, F), jnp.float32),
        mesh=_MESH,
        scratch_types=[
            pltpu.VMEM((CH, K), jnp.int32),
            pltpu.VMEM((CH, K), jnp.int32),
            pltpu.VMEM((K, F), jnp.float32),
            pltpu.VMEM_SHARED((NP, F), jnp.float32),
            pltpu.SemaphoreType.DMA,
        ],
        compiler_params=_SC_PARAMS,
    )
    def spmm(y_hbm, src_hbm, dst_hbm, zeros_hbm, out_hbm, src_v, dst_v, rows_v, zsh, sem):
        c = lax.axis_index("c")
        s = lax.axis_index("s")
        w = s * NC + c
        pltpu.sync_copy(src_hbm.at[pl.ds(w * CH, CH)], src_v)
        pltpu.sync_copy(dst_hbm.at[pl.ds(w * CH, CH)], dst_v)
        pltpu.sync_copy(zeros_hbm.at[pl.ds(s * RP, RP)], zsh.at[pl.ds(s * RP, RP)])
        plsc.subcore_barrier()

        def step(j, carry):
            pltpu.async_copy(y_hbm.at[src_v.at[j]], rows_v, sem).wait()
            pltpu.sync_copy(rows_v, zsh.at[dst_v.at[j]], add=True)
            return carry

        lax.fori_loop(0, CH, step, 0)
        plsc.subcore_barrier()
        pltpu.sync_copy(zsh.at[pl.ds(s * RP, RP)], out_hbm.at[c].at[pl.ds(s * RP, RP)])

    return spmm(y, src, dst, zeros)


@jax.jit
def _headgather(hw, h, src, dst):
    HD = 32

    @functools.partial(
        pl.kernel,
        out_type=(
            jax.ShapeDtypeStruct((E, HD), jnp.float32),
            jax.ShapeDtypeStruct((E, HD), jnp.float32),
        ),
        mesh=_MESH,
        scratch_types=[
            pltpu.VMEM((CH, K), jnp.int32),
            pltpu.VMEM((CH, K), jnp.int32),
            pltpu.VMEM((K, HD), jnp.float32),
            pltpu.VMEM((K, HD), jnp.float32),
            pltpu.SemaphoreType.DMA,
            pltpu.SemaphoreType.DMA,
        ],
        compiler_params=_SC_PARAMS,
    )
    def gat(hw_hbm, h_hbm, src_hbm, dst_hbm, oa_hbm, ob_hbm, src_v, dst_v, ra_v, rb_v, sa, sb):
        c = lax.axis_index("c")
        s = lax.axis_index("s")
        w = s * NC + c
        pltpu.sync_copy(src_hbm.at[pl.ds(w * CH, CH)], src_v)
        pltpu.sync_copy(dst_hbm.at[pl.ds(w * CH, CH)], dst_v)

        def step(j, carry):
            ca = pltpu.async_copy(hw_hbm.at[src_v.at[j]], ra_v, sa)
            cb = pltpu.async_copy(h_hbm.at[dst_v.at[j]], rb_v, sb)
            ca.wait()
            cb.wait()
            base = w * (E // NW) + j * K
            pltpu.sync_copy(ra_v, oa_hbm.at[pl.ds(base, K)])
            pltpu.sync_copy(rb_v, ob_hbm.at[pl.ds(base, K)])
            return carry

        lax.fori_loop(0, CH, step, 0)

    return gat(hw, h, src, dst)


def _first_tc(x, W0, degp):
    F = W0.shape[1]

    def body(x_ref, w_ref, dp_ref, y_ref, dinv_ref):
        deg = dp_ref[0, :, 0:1] + dp_ref[1, :, 0:1] + 1.0
        dinv = lax.rsqrt(deg)
        dinv_ref[...] = dinv
        y_ref[...] = dinv * jnp.dot(x_ref[...], w_ref[...], preferred_element_type=jnp.float32)

    return pl.pallas_call(
        body,
        grid=(NB,),
        in_specs=[
            pl.BlockSpec((RB, x.shape[1]), lambda b: (b, 0)),
            pl.BlockSpec((x.shape[1], F), lambda b: (0, 0)),
            pl.BlockSpec((2, RB, 8), lambda b: (0, b, 0)),
        ],
        out_specs=(
            pl.BlockSpec((RB, F), lambda b: (b, 0)),
            pl.BlockSpec((RB, 1), lambda b: (b, 0)),
        ),
        out_shape=(
            jax.ShapeDtypeStruct((N, F), jnp.float32),
            jax.ShapeDtypeStruct((N, 1), jnp.float32),
        ),
    )(x, W0, degp)


def _stats_tc(z, y, dinv, b):
    F = y.shape[1]

    def body(z_ref, y_ref, dinv_ref, b_ref, t_ref, st_ref):
        i = pl.program_id(0)
        t = dinv_ref[...] * (z_ref[0] + z_ref[1] + y_ref[...]) + b_ref[...]
        t_ref[...] = t

        @pl.when(i == 0)
        def _():
            st_ref[...] = jnp.zeros_like(st_ref)

        st_ref[0:1, :] += jnp.sum(t, axis=0, keepdims=True)
        st_ref[1:2, :] += jnp.sum(t * t, axis=0, keepdims=True)

    return pl.pallas_call(
        body,
        grid=(NB,),
        in_specs=[
            pl.BlockSpec((2, RB, F), lambda b: (0, b, 0)),
            pl.BlockSpec((RB, F), lambda b: (b, 0)),
            pl.BlockSpec((RB, 1), lambda b: (b, 0)),
            pl.BlockSpec((1, F), lambda b: (0, 0)),
        ],
        out_specs=(
            pl.BlockSpec((RB, F), lambda b: (b, 0)),
            pl.BlockSpec((2, F), lambda b: (0, 0)),
        ),
        out_shape=(
            jax.ShapeDtypeStruct((N, F), jnp.float32),
            jax.ShapeDtypeStruct((2, F), jnp.float32),
        ),
    )(z, y, dinv, b)


def _bn_relu(t_ref, st_ref, g_ref, be_ref):
    mu = st_ref[0:1, :] * (1.0 / N)
    var = st_ref[1:2, :] * (1.0 / N) - mu * mu
    inv = lax.rsqrt(var + 1e-5)
    return jnp.maximum(g_ref[...] * (t_ref[...] - mu) * inv + be_ref[...], 0.0)


def _apply_tc(t, st, g, be, dinv, W):
    F = t.shape[1]
    F2 = W.shape[1]

    def body(t_ref, st_ref, g_ref, be_ref, dinv_ref, w_ref, o_ref):
        h = _bn_relu(t_ref, st_ref, g_ref, be_ref)
        o_ref[...] = dinv_ref[...] * jnp.dot(h, w_ref[...], preferred_element_type=jnp.float32)

    return pl.pallas_call(
        body,
        grid=(NB,),
        in_specs=[
            pl.BlockSpec((RB, F), lambda b: (b, 0)),
            pl.BlockSpec((2, F), lambda b: (0, 0)),
            pl.BlockSpec((1, F), lambda b: (0, 0)),
            pl.BlockSpec((1, F), lambda b: (0, 0)),
            pl.BlockSpec((RB, 1), lambda b: (b, 0)),
            pl.BlockSpec((F, F2), lambda b: (0, 0)),
        ],
        out_specs=pl.BlockSpec((RB, F2), lambda b: (b, 0)),
        out_shape=jax.ShapeDtypeStruct((N, F2), jnp.float32),
    )(t, st, g, be, dinv, W)


def _apply_last_tc(t, st, g, be, fcw_row):
    F = t.shape[1]

    def body(t_ref, st_ref, g_ref, be_ref, fw_ref, h_ref, hw_ref):
        h = _bn_relu(t_ref, st_ref, g_ref, be_ref)
        h_ref[...] = h
        hw_ref[...] = h * fw_ref[...]

    return pl.pallas_call(
        body,
        grid=(NB,),
        in_specs=[
            pl.BlockSpec((RB, F), lambda b: (b, 0)),
            pl.BlockSpec((2, F), lambda b: (0, 0)),
            pl.BlockSpec((1, F), lambda b: (0, 0)),
            pl.BlockSpec((1, F), lambda b: (0, 0)),
            pl.BlockSpec((1, F), lambda b: (0, 0)),
        ],
        out_specs=(
            pl.BlockSpec((RB, F), lambda b: (b, 0)),
            pl.BlockSpec((RB, F), lambda b: (b, 0)),
        ),
        out_shape=(
            jax.ShapeDtypeStruct((N, F), jnp.float32),
            jax.ShapeDtypeStruct((N, F), jnp.float32),
        ),
    )(t, st, g, be, fcw_row)


def _head_tc(a, b, fcb):
    BE = 8000

    def body(a_ref, b_ref, c_ref, o_ref):
        r = jnp.sum(a_ref[...] * b_ref[...], axis=1, keepdims=True) + c_ref[...]
        o_ref[...] = 1.0 / (1.0 + jnp.exp(-r))

    return pl.pallas_call(
        body,
        grid=(E // BE,),
        in_specs=[
            pl.BlockSpec((BE, 32), lambda i: (i, 0)),
            pl.BlockSpec((BE, 32), lambda i: (i, 0)),
            pl.BlockSpec((1, 1), lambda i: (0, 0)),
        ],
        out_specs=pl.BlockSpec((BE, 1), lambda i: (i, 0)),
        out_shape=jax.ShapeDtypeStruct((E, 1), jnp.float32),
    )(a, b, fcb)


def kernel(x, edge_index, W0, b0, g0, be0, W1, b1, g1, be1, W2, b2, g2, be2,
           W3, b3, g3, be3, W4, b4, g4, be4, fcW, fcb):
    src = edge_index[0].reshape(E // K, K)
    dst = edge_index[1].reshape(E // K, K)
    Ws = [W0, W1, W2, W3, W4]
    bs = [b0, b1, b2, b3, b4]
    gs = [g0, g1, g2, g3, g4]
    bes = [be0, be1, be2, be3, be4]

    degp = _spmm(jnp.ones((N, 8), jnp.float32), src, dst, jnp.zeros((NP, 8), jnp.float32), 8)
    y, dinv = _first_tc(x, W0, degp)

    for i in range(5):
        F = y.shape[1]
        if F > 128:
            zl = _spmm(y[:, :128], src, dst, jnp.zeros((NP, 128), jnp.float32), 128)
            zr = _spmm(y[:, 128:], src, dst, jnp.zeros((NP, 128), jnp.float32), 128)
            z = jnp.concatenate([zl, zr], axis=2)
        else:
            z = _spmm(y, src, dst, jnp.zeros((NP, F), jnp.float32), F)
        t, st = _stats_tc(z, y, dinv, bs[i].reshape(1, F))
        if i < 4:
            y = _apply_tc(t, st, gs[i].reshape(1, F), bes[i].reshape(1, F), dinv, Ws[i + 1])
        else:
            h5, h5w = _apply_last_tc(t, st, gs[i].reshape(1, F), bes[i].reshape(1, F),
                                     fcW[:, 0].reshape(1, F))

    asrc, bdst = _headgather(h5w, h5, src, dst)
    return _head_tc(asrc, bdst, fcb.reshape(1, 1))


# speedup vs baseline: 15.3259x; 1.4434x over previous
import functools

import jax
import jax.numpy as jnp
from jax import lax
from jax.experimental import pallas as pl
from jax.experimental.pallas import tpu as pltpu
from jax.experimental.pallas import tpu_sc as plsc

N = 10000
NP = 10240
E = 320000
NC = 2
NS = 16
NW = NC * NS
K = 100
CH = E // (NW * K)
RP = NP // NS
RB = 1000
NB = N // RB

_MESH = plsc.VectorSubcoreMesh(core_axis_name="c", subcore_axis_name="s")
_SC_PARAMS = pltpu.CompilerParams(use_tc_tiling_on_sc=False)
_SC_VPARAMS = pltpu.CompilerParams(use_tc_tiling_on_sc=False, needs_layout_passes=False)


@functools.partial(jax.jit, static_argnums=(4,))
def _spmm(y, src, dst, zeros, F):
    @functools.partial(
        pl.kernel,
        out_type=jax.ShapeDtypeStruct((NC, NP, F), jnp.float32),
        mesh=_MESH,
        scratch_types=[
            pltpu.VMEM((CH, K), jnp.int32),
            pltpu.VMEM((CH, K), jnp.int32),
            pltpu.VMEM((K, F), jnp.float32),
            pltpu.VMEM((K, F), jnp.float32),
            pltpu.VMEM_SHARED((NP, F), jnp.float32),
            pltpu.SemaphoreType.DMA,
            pltpu.SemaphoreType.DMA,
        ],
        compiler_params=_SC_PARAMS,
    )
    def spmm(y_hbm, src_hbm, dst_hbm, zeros_hbm, out_hbm, src_v, dst_v, rows0_v, rows1_v, zsh, sem0, sem1):
        c = lax.axis_index("c")
        s = lax.axis_index("s")
        w = s * NC + c
        pltpu.sync_copy(src_hbm.at[pl.ds(w * CH, CH)], src_v)
        pltpu.sync_copy(dst_hbm.at[pl.ds(w * CH, CH)], dst_v)
        pltpu.sync_copy(zeros_hbm.at[pl.ds(s * RP, RP)], zsh.at[pl.ds(s * RP, RP)])
        pltpu.async_copy(y_hbm.at[src_v.at[0]], rows0_v, sem0)
        pltpu.async_copy(y_hbm.at[src_v.at[1]], rows1_v, sem1)
        plsc.subcore_barrier()

        def step(jj, carry):
            j0 = jj * 2
            pltpu.make_async_copy(y_hbm.at[src_v.at[j0]], rows0_v, sem0).wait()
            pltpu.sync_copy(rows0_v, zsh.at[dst_v.at[j0]], add=True)
            pltpu.async_copy(y_hbm.at[src_v.at[jnp.minimum(j0 + 2, CH - 1)]], rows0_v, sem0)
            pltpu.make_async_copy(y_hbm.at[src_v.at[j0 + 1]], rows1_v, sem1).wait()
            pltpu.sync_copy(rows1_v, zsh.at[dst_v.at[j0 + 1]], add=True)
            pltpu.async_copy(y_hbm.at[src_v.at[jnp.minimum(j0 + 3, CH - 1)]], rows1_v, sem1)
            return carry

        lax.fori_loop(0, CH // 2, step, 0)
        pltpu.make_async_copy(y_hbm.at[src_v.at[CH - 1]], rows0_v, sem0).wait()
        pltpu.make_async_copy(y_hbm.at[src_v.at[CH - 1]], rows1_v, sem1).wait()
        plsc.subcore_barrier()
        pltpu.sync_copy(zsh.at[pl.ds(s * RP, RP)], out_hbm.at[c].at[pl.ds(s * RP, RP)])

    return spmm(y, src, dst, zeros)


KH = 80
CHH = E // (NW * KH)
EPW = E // NW


@jax.jit
def _headdot(hw, h, src, dst, fcb16):
    HD = 32

    @functools.partial(
        pl.kernel,
        out_type=jax.ShapeDtypeStruct((E,), jnp.float32),
        mesh=_MESH,
        scratch_types=[
            pltpu.VMEM((CHH, KH), jnp.int32),
            pltpu.VMEM((CHH, KH), jnp.int32),
            pltpu.VMEM((KH, HD), jnp.float32),
            pltpu.VMEM((KH, HD), jnp.float32),
            pltpu.VMEM((KH, HD), jnp.float32),
            pltpu.VMEM((KH, HD), jnp.float32),
            pltpu.VMEM((16,), jnp.float32),
            pltpu.VMEM((EPW,), jnp.float32),
            pltpu.SemaphoreType.DMA,
            pltpu.SemaphoreType.DMA,
        ],
        compiler_params=_SC_VPARAMS,
    )
    def hd(hw_hbm, h_hbm, src_hbm, dst_hbm, fcb_hbm, out_hbm, src_v, dst_v,
           ra0_v, rb0_v, ra1_v, rb1_v, fcb_v, out_v, s0, s1):
        c = lax.axis_index("c")
        s = lax.axis_index("s")
        w = s * NC + c
        pltpu.sync_copy(src_hbm.at[pl.ds(w * CHH, CHH)], src_v)
        pltpu.sync_copy(dst_hbm.at[pl.ds(w * CHH, CHH)], dst_v)
        pltpu.sync_copy(fcb_hbm, fcb_v)
        pltpu.async_copy(hw_hbm.at[src_v.at[0]], ra0_v, s0)
        pltpu.async_copy(h_hbm.at[dst_v.at[0]], rb0_v, s0)
        pltpu.async_copy(hw_hbm.at[src_v.at[1]], ra1_v, s1)
        pltpu.async_copy(h_hbm.at[dst_v.at[1]], rb1_v, s1)
        lanes = jnp.arange(16, dtype=jnp.int32)

        def process(j, ra_v, rb_v, sem, prefetch):
            pltpu.make_async_copy(hw_hbm.at[src_v.at[j]], ra_v, sem).wait()
            pltpu.make_async_copy(h_hbm.at[dst_v.at[j]], rb_v, sem).wait()
            for g in range(KH // 16):
                rows = lanes + (g * 16)
                acc = fcb_v[...]
                for k in range(HD):
                    col = jnp.full((16,), k, jnp.int32)
                    acc = acc + plsc.load_gather(ra_v, [rows, col]) * plsc.load_gather(rb_v, [rows, col])
                out_v[pl.ds(j * KH + g * 16, 16)] = 1.0 / (1.0 + jnp.exp(-acc))
            if prefetch:
                jp = jnp.minimum(j + 2, CHH - 1)
                pltpu.async_copy(hw_hbm.at[src_v.at[jp]], ra_v, sem)
                pltpu.async_copy(h_hbm.at[dst_v.at[jp]], rb_v, sem)

        def step(jj, carry):
            process(jj * 2, ra0_v, rb0_v, s0, True)
            process(jj * 2 + 1, ra1_v, rb1_v, s1, True)
            return carry

        lax.fori_loop(0, CHH // 2, step, 0)
        process(CHH - 1, ra0_v, rb0_v, s0, False)
        pltpu.make_async_copy(hw_hbm.at[src_v.at[0]], ra1_v, s1).wait()
        pltpu.make_async_copy(h_hbm.at[dst_v.at[0]], rb1_v, s1).wait()
        pltpu.sync_copy(out_v, out_hbm.at[pl.ds(w * EPW, EPW)])

    return hd(hw, h, src, dst, fcb16)


def _first_tc(x, W0, degp):
    F = W0.shape[1]

    def body(x_ref, w_ref, dp_ref, y_ref, dinv_ref):
        deg = dp_ref[0, :, 0:1] + dp_ref[1, :, 0:1] + 1.0
        dinv = lax.rsqrt(deg)
        dinv_ref[...] = dinv
        y_ref[...] = dinv * jnp.dot(x_ref[...], w_ref[...], preferred_element_type=jnp.float32)

    return pl.pallas_call(
        body,
        grid=(NB,),
        in_specs=[
            pl.BlockSpec((RB, x.shape[1]), lambda b: (b, 0)),
            pl.BlockSpec((x.shape[1], F), lambda b: (0, 0)),
            pl.BlockSpec((2, RB, 8), lambda b: (0, b, 0)),
        ],
        out_specs=(
            pl.BlockSpec((RB, F), lambda b: (b, 0)),
            pl.BlockSpec((RB, 1), lambda b: (b, 0)),
        ),
        out_shape=(
            jax.ShapeDtypeStruct((N, F), jnp.float32),
            jax.ShapeDtypeStruct((N, 1), jnp.float32),
        ),
    )(x, W0, degp)


def _stats_tc(z, y, dinv, b):
    F = y.shape[1]

    def body(z_ref, y_ref, dinv_ref, b_ref, t_ref, st_ref):
        i = pl.program_id(0)
        t = dinv_ref[...] * (z_ref[0] + z_ref[1] + y_ref[...]) + b_ref[...]
        t_ref[...] = t

        @pl.when(i == 0)
        def _():
            st_ref[...] = jnp.zeros_like(st_ref)

        st_ref[0:1, :] += jnp.sum(t, axis=0, keepdims=True)
        st_ref[1:2, :] += jnp.sum(t * t, axis=0, keepdims=True)

    return pl.pallas_call(
        body,
        grid=(NB,),
        in_specs=[
            pl.BlockSpec((2, RB, F), lambda b: (0, b, 0)),
            pl.BlockSpec((RB, F), lambda b: (b, 0)),
            pl.BlockSpec((RB, 1), lambda b: (b, 0)),
            pl.BlockSpec((1, F), lambda b: (0, 0)),
        ],
        out_specs=(
            pl.BlockSpec((RB, F), lambda b: (b, 0)),
            pl.BlockSpec((2, F), lambda b: (0, 0)),
        ),
        out_shape=(
            jax.ShapeDtypeStruct((N, F), jnp.float32),
            jax.ShapeDtypeStruct((2, F), jnp.float32),
        ),
    )(z, y, dinv, b)


def _bn_relu(t_ref, st_ref, g_ref, be_ref):
    mu = st_ref[0:1, :] * (1.0 / N)
    var = st_ref[1:2, :] * (1.0 / N) - mu * mu
    inv = lax.rsqrt(var + 1e-5)
    return jnp.maximum(g_ref[...] * (t_ref[...] - mu) * inv + be_ref[...], 0.0)


def _apply_tc(t, st, g, be, dinv, W):
    F = t.shape[1]
    F2 = W.shape[1]

    def body(t_ref, st_ref, g_ref, be_ref, dinv_ref, w_ref, o_ref):
        h = _bn_relu(t_ref, st_ref, g_ref, be_ref)
        o_ref[...] = dinv_ref[...] * jnp.dot(h, w_ref[...], preferred_element_type=jnp.float32)

    return pl.pallas_call(
        body,
        grid=(NB,),
        in_specs=[
            pl.BlockSpec((RB, F), lambda b: (b, 0)),
            pl.BlockSpec((2, F), lambda b: (0, 0)),
            pl.BlockSpec((1, F), lambda b: (0, 0)),
            pl.BlockSpec((1, F), lambda b: (0, 0)),
            pl.BlockSpec((RB, 1), lambda b: (b, 0)),
            pl.BlockSpec((F, F2), lambda b: (0, 0)),
        ],
        out_specs=pl.BlockSpec((RB, F2), lambda b: (b, 0)),
        out_shape=jax.ShapeDtypeStruct((N, F2), jnp.float32),
    )(t, st, g, be, dinv, W)


def _apply_last_tc(t, st, g, be, fcw_row):
    F = t.shape[1]

    def body(t_ref, st_ref, g_ref, be_ref, fw_ref, h_ref, hw_ref):
        h = _bn_relu(t_ref, st_ref, g_ref, be_ref)
        h_ref[...] = h
        hw_ref[...] = h * fw_ref[...]

    return pl.pallas_call(
        body,
        grid=(NB,),
        in_specs=[
            pl.BlockSpec((RB, F), lambda b: (b, 0)),
            pl.BlockSpec((2, F), lambda b: (0, 0)),
            pl.BlockSpec((1, F), lambda b: (0, 0)),
            pl.BlockSpec((1, F), lambda b: (0, 0)),
            pl.BlockSpec((1, F), lambda b: (0, 0)),
        ],
        out_specs=(
            pl.BlockSpec((RB, F), lambda b: (b, 0)),
            pl.BlockSpec((RB, F), lambda b: (b, 0)),
        ),
        out_shape=(
            jax.ShapeDtypeStruct((N, F), jnp.float32),
            jax.ShapeDtypeStruct((N, F), jnp.float32),
        ),
    )(t, st, g, be, fcw_row)


def kernel(x, edge_index, W0, b0, g0, be0, W1, b1, g1, be1, W2, b2, g2, be2,
           W3, b3, g3, be3, W4, b4, g4, be4, fcW, fcb):
    src = edge_index[0].reshape(E // K, K)
    dst = edge_index[1].reshape(E // K, K)
    Ws = [W0, W1, W2, W3, W4]
    bs = [b0, b1, b2, b3, b4]
    gs = [g0, g1, g2, g3, g4]
    bes = [be0, be1, be2, be3, be4]

    degp = _spmm(jnp.ones((N, 8), jnp.float32), src, dst, jnp.zeros((NP, 8), jnp.float32), 8)
    y, dinv = _first_tc(x, W0, degp)

    for i in range(5):
        F = y.shape[1]
        if F > 128:
            zl = _spmm(y[:, :128], src, dst, jnp.zeros((NP, 128), jnp.float32), 128)
            zr = _spmm(y[:, 128:], src, dst, jnp.zeros((NP, 128), jnp.float32), 128)
            z = jnp.concatenate([zl, zr], axis=2)
        else:
            z = _spmm(y, src, dst, jnp.zeros((NP, F), jnp.float32), F)
        t, st = _stats_tc(z, y, dinv, bs[i].reshape(1, F))
        if i < 4:
            y = _apply_tc(t, st, gs[i].reshape(1, F), bes[i].reshape(1, F), dinv, Ws[i + 1])
        else:
            h5, h5w = _apply_last_tc(t, st, gs[i].reshape(1, F), bes[i].reshape(1, F),
                                     fcW[:, 0].reshape(1, F))

    srch = edge_index[0].reshape(E // KH, KH)
    dsth = edge_index[1].reshape(E // KH, KH)
    fcb16 = jnp.broadcast_to(fcb.reshape(1), (16,))
    return _headdot(h5w, h5, srch, dsth, fcb16).reshape(E, 1)


# speedup vs baseline: 16.1088x; 1.0511x over previous
import functools

import jax
import jax.numpy as jnp
from jax import lax
from jax.experimental import pallas as pl
from jax.experimental.pallas import tpu as pltpu
from jax.experimental.pallas import tpu_sc as plsc

N = 10000
NP = 10240
E = 320000
NC = 2
NS = 16
NW = NC * NS
K = 100
CH = E // (NW * K)
RP = NP // NS
RB = 1000
NB = N // RB

_MESH = plsc.VectorSubcoreMesh(core_axis_name="c", subcore_axis_name="s")
_SC_PARAMS = pltpu.CompilerParams(use_tc_tiling_on_sc=False)
_SC_VPARAMS = pltpu.CompilerParams(use_tc_tiling_on_sc=False, needs_layout_passes=False)


@functools.partial(jax.jit, static_argnums=(3, 4))
def _spmm(y, e3, zeros, NFC, FC):
    @functools.partial(
        pl.kernel,
        out_type=jax.ShapeDtypeStruct((NC, NFC, NP, FC), jnp.float32),
        mesh=_MESH,
        scratch_types=[
            pltpu.VMEM((CH, K), jnp.int32),
            pltpu.VMEM((CH, K), jnp.int32),
            pltpu.VMEM((K, FC), jnp.float32),
            pltpu.VMEM((K, FC), jnp.float32),
            pltpu.VMEM_SHARED((NP, FC), jnp.float32),
            pltpu.SemaphoreType.DMA,
            pltpu.SemaphoreType.DMA,
        ],
        compiler_params=_SC_PARAMS,
    )
    def spmm(y_hbm, e3_hbm, zeros_hbm, out_hbm, src_v, dst_v, rows0_v, rows1_v, zsh, sem0, sem1):
        c = lax.axis_index("c")
        s = lax.axis_index("s")
        w = s * NC + c
        pltpu.sync_copy(e3_hbm.at[0].at[pl.ds(w * CH, CH)], src_v)
        pltpu.sync_copy(e3_hbm.at[1].at[pl.ds(w * CH, CH)], dst_v)
        for f in range(NFC):
            yf = y_hbm.at[f]
            pltpu.sync_copy(zeros_hbm.at[pl.ds(s * RP, RP)], zsh.at[pl.ds(s * RP, RP)])
            pltpu.async_copy(yf.at[src_v.at[0]], rows0_v, sem0)
            pltpu.async_copy(yf.at[src_v.at[1]], rows1_v, sem1)
            plsc.subcore_barrier()

            def step(jj, carry):
                j0 = jj * 2
                pltpu.make_async_copy(yf.at[src_v.at[j0]], rows0_v, sem0).wait()
                pltpu.sync_copy(rows0_v, zsh.at[dst_v.at[j0]], add=True)
                pltpu.async_copy(yf.at[src_v.at[jnp.minimum(j0 + 2, CH - 1)]], rows0_v, sem0)
                pltpu.make_async_copy(yf.at[src_v.at[j0 + 1]], rows1_v, sem1).wait()
                pltpu.sync_copy(rows1_v, zsh.at[dst_v.at[j0 + 1]], add=True)
                pltpu.async_copy(yf.at[src_v.at[jnp.minimum(j0 + 3, CH - 1)]], rows1_v, sem1)
                return carry

            lax.fori_loop(0, CH // 2, step, 0)
            pltpu.make_async_copy(yf.at[src_v.at[CH - 1]], rows0_v, sem0).wait()
            pltpu.make_async_copy(yf.at[src_v.at[CH - 1]], rows1_v, sem1).wait()
            plsc.subcore_barrier()
            pltpu.sync_copy(zsh.at[pl.ds(s * RP, RP)],
                            out_hbm.at[c].at[f].at[pl.ds(s * RP, RP)])

    return spmm(y, e3, zeros)


@jax.jit
def _deg(e3, ones, zeros):
    @functools.partial(
        pl.kernel,
        out_type=jax.ShapeDtypeStruct((NC, NP, 8), jnp.float32),
        mesh=_MESH,
        scratch_types=[
            pltpu.VMEM((CH, K), jnp.int32),
            pltpu.VMEM((K, 8), jnp.float32),
            pltpu.VMEM_SHARED((NP, 8), jnp.float32),
        ],
        compiler_params=_SC_PARAMS,
    )
    def deg(e3_hbm, ones_hbm, zeros_hbm, out_hbm, dst_v, ones_v, zsh):
        c = lax.axis_index("c")
        s = lax.axis_index("s")
        w = s * NC + c
        pltpu.sync_copy(e3_hbm.at[1].at[pl.ds(w * CH, CH)], dst_v)
        pltpu.sync_copy(ones_hbm, ones_v)
        pltpu.sync_copy(zeros_hbm.at[pl.ds(s * RP, RP)], zsh.at[pl.ds(s * RP, RP)])
        plsc.subcore_barrier()

        def step(j, carry):
            pltpu.sync_copy(ones_v, zsh.at[dst_v.at[j]], add=True)
            return carry

        lax.fori_loop(0, CH, step, 0)
        plsc.subcore_barrier()
        pltpu.sync_copy(zsh.at[pl.ds(s * RP, RP)], out_hbm.at[c].at[pl.ds(s * RP, RP)])

    return deg(e3, ones, zeros)


KH = 80
CHH = E // (NW * KH)
EPW = E // NW


@jax.jit
def _headdot(hw, h, e3h, fcb16):
    HD = 32

    @functools.partial(
        pl.kernel,
        out_type=jax.ShapeDtypeStruct((E,), jnp.float32),
        mesh=_MESH,
        scratch_types=[
            pltpu.VMEM((CHH, KH), jnp.int32),
            pltpu.VMEM((CHH, KH), jnp.int32),
            pltpu.VMEM((KH, HD), jnp.float32),
            pltpu.VMEM((KH, HD), jnp.float32),
            pltpu.VMEM((KH, HD), jnp.float32),
            pltpu.VMEM((KH, HD), jnp.float32),
            pltpu.VMEM((16,), jnp.float32),
            pltpu.VMEM((EPW,), jnp.float32),
            pltpu.SemaphoreType.DMA,
            pltpu.SemaphoreType.DMA,
        ],
        compiler_params=_SC_VPARAMS,
    )
    def hd(hw_hbm, h_hbm, e3_hbm, fcb_hbm, out_hbm, src_v, dst_v,
           ra0_v, rb0_v, ra1_v, rb1_v, fcb_v, out_v, s0, s1):
        c = lax.axis_index("c")
        s = lax.axis_index("s")
        w = s * NC + c
        pltpu.sync_copy(e3_hbm.at[0].at[pl.ds(w * CHH, CHH)], src_v)
        pltpu.sync_copy(e3_hbm.at[1].at[pl.ds(w * CHH, CHH)], dst_v)
        pltpu.sync_copy(fcb_hbm, fcb_v)
        pltpu.async_copy(hw_hbm.at[src_v.at[0]], ra0_v, s0)
        pltpu.async_copy(h_hbm.at[dst_v.at[0]], rb0_v, s0)
        pltpu.async_copy(hw_hbm.at[src_v.at[1]], ra1_v, s1)
        pltpu.async_copy(h_hbm.at[dst_v.at[1]], rb1_v, s1)
        lanes = jnp.arange(16, dtype=jnp.int32)

        def process(j, ra_v, rb_v, sem, prefetch):
            pltpu.make_async_copy(hw_hbm.at[src_v.at[j]], ra_v, sem).wait()
            pltpu.make_async_copy(h_hbm.at[dst_v.at[j]], rb_v, sem).wait()
            for g in range(KH // 16):
                rows = lanes + (g * 16)
                accs = [fcb_v[...], None, None, None]
                for k in range(HD):
                    col = jnp.full((16,), k, jnp.int32)
                    p = plsc.load_gather(ra_v, [rows, col]) * plsc.load_gather(rb_v, [rows, col])
                    a = accs[k % 4]
                    accs[k % 4] = p if a is None else a + p
                acc = (accs[0] + accs[1]) + (accs[2] + accs[3])
                out_v[pl.ds(j * KH + g * 16, 16)] = 1.0 / (1.0 + jnp.exp(-acc))
            if prefetch:
                jp = jnp.minimum(j + 2, CHH - 1)
                pltpu.async_copy(hw_hbm.at[src_v.at[jp]], ra_v, sem)
                pltpu.async_copy(h_hbm.at[dst_v.at[jp]], rb_v, sem)

        def step(jj, carry):
            process(jj * 2, ra0_v, rb0_v, s0, True)
            process(jj * 2 + 1, ra1_v, rb1_v, s1, True)
            return carry

        lax.fori_loop(0, CHH // 2, step, 0)
        process(CHH - 1, ra0_v, rb0_v, s0, False)
        pltpu.make_async_copy(hw_hbm.at[src_v.at[0]], ra1_v, s1).wait()
        pltpu.make_async_copy(h_hbm.at[dst_v.at[0]], rb1_v, s1).wait()
        pltpu.sync_copy(out_v, out_hbm.at[pl.ds(w * EPW, EPW)])

    return hd(hw, h, e3h, fcb16)


def _first_tc(x, W0, degp):
    F = W0.shape[1]

    def body(x_ref, w_ref, dp_ref, y_ref, dinv_ref):
        deg = dp_ref[0, :, 0:1] + dp_ref[1, :, 0:1] + 1.0
        dinv = lax.rsqrt(deg)
        dinv_ref[...] = dinv
        y_ref[0] = dinv * jnp.dot(x_ref[...], w_ref[...], preferred_element_type=jnp.float32)

    return pl.pallas_call(
        body,
        grid=(NB,),
        in_specs=[
            pl.BlockSpec((RB, x.shape[1]), lambda b: (b, 0)),
            pl.BlockSpec((x.shape[1], F), lambda b: (0, 0)),
            pl.BlockSpec((2, RB, 8), lambda b: (0, b, 0)),
        ],
        out_specs=(
            pl.BlockSpec((1, RB, F), lambda b: (0, b, 0)),
            pl.BlockSpec((RB, 1), lambda b: (b, 0)),
        ),
        out_shape=(
            jax.ShapeDtypeStruct((1, N, F), jnp.float32),
            jax.ShapeDtypeStruct((N, 1), jnp.float32),
        ),
    )(x, W0, degp)


def _stats_tc(z, y, dinv, b):
    NFC, _, FC = y.shape
    F = NFC * FC

    def body(z_ref, y_ref, dinv_ref, b_ref, t_ref, st_ref):
        i = pl.program_id(0)
        parts = [z_ref[0, f] + z_ref[1, f] + y_ref[f] for f in range(NFC)]
        zy = jnp.concatenate(parts, axis=1) if NFC > 1 else parts[0]
        t = dinv_ref[...] * zy + b_ref[...]
        t_ref[...] = t

        @pl.when(i == 0)
        def _():
            st_ref[...] = jnp.zeros_like(st_ref)

        st_ref[0:1, :] += jnp.sum(t, axis=0, keepdims=True)
        st_ref[1:2, :] += jnp.sum(t * t, axis=0, keepdims=True)

    return pl.pallas_call(
        body,
        grid=(NB,),
        in_specs=[
            pl.BlockSpec((2, NFC, RB, FC), lambda b: (0, 0, b, 0)),
            pl.BlockSpec((NFC, RB, FC), lambda b: (0, b, 0)),
            pl.BlockSpec((RB, 1), lambda b: (b, 0)),
            pl.BlockSpec((1, F), lambda b: (0, 0)),
        ],
        out_specs=(
            pl.BlockSpec((RB, F), lambda b: (b, 0)),
            pl.BlockSpec((2, F), lambda b: (0, 0)),
        ),
        out_shape=(
            jax.ShapeDtypeStruct((N, F), jnp.float32),
            jax.ShapeDtypeStruct((2, F), jnp.float32),
        ),
    )(z, y, dinv, b)


def _bn_relu(t_ref, st_ref, g_ref, be_ref):
    mu = st_ref[0:1, :] * (1.0 / N)
    var = st_ref[1:2, :] * (1.0 / N) - mu * mu
    inv = lax.rsqrt(var + 1e-5)
    return jnp.maximum(g_ref[...] * (t_ref[...] - mu) * inv + be_ref[...], 0.0)


def _apply_tc(t, st, g, be, dinv, W, NFC2, FC2):
    F = t.shape[1]

    def body(t_ref, st_ref, g_ref, be_ref, dinv_ref, w_ref, o_ref):
        h = _bn_relu(t_ref, st_ref, g_ref, be_ref)
        yp = dinv_ref[...] * jnp.dot(h, w_ref[...], preferred_element_type=jnp.float32)
        for f in range(NFC2):
            o_ref[f] = yp[:, f * FC2:(f + 1) * FC2]

    return pl.pallas_call(
        body,
        grid=(NB,),
        in_specs=[
            pl.BlockSpec((RB, F), lambda b: (b, 0)),
            pl.BlockSpec((2, F), lambda b: (0, 0)),
            pl.BlockSpec((1, F), lambda b: (0, 0)),
            pl.BlockSpec((1, F), lambda b: (0, 0)),
            pl.BlockSpec((RB, 1), lambda b: (b, 0)),
            pl.BlockSpec((F, NFC2 * FC2), lambda b: (0, 0)),
        ],
        out_specs=pl.BlockSpec((NFC2, RB, FC2), lambda b: (0, b, 0)),
        out_shape=jax.ShapeDtypeStruct((NFC2, N, FC2), jnp.float32),
    )(t, st, g, be, dinv, W)


def _apply_last_tc(t, st, g, be, fcw_row):
    F = t.shape[1]

    def body(t_ref, st_ref, g_ref, be_ref, fw_ref, h_ref, hw_ref):
        h = _bn_relu(t_ref, st_ref, g_ref, be_ref)
        h_ref[...] = h
        hw_ref[...] = h * fw_ref[...]

    return pl.pallas_call(
        body,
        grid=(NB,),
        in_specs=[
            pl.BlockSpec((RB, F), lambda b: (b, 0)),
            pl.BlockSpec((2, F), lambda b: (0, 0)),
            pl.BlockSpec((1, F), lambda b: (0, 0)),
            pl.BlockSpec((1, F), lambda b: (0, 0)),
            pl.BlockSpec((1, F), lambda b: (0, 0)),
        ],
        out_specs=(
            pl.BlockSpec((RB, F), lambda b: (b, 0)),
            pl.BlockSpec((RB, F), lambda b: (b, 0)),
        ),
        out_shape=(
            jax.ShapeDtypeStruct((N, F), jnp.float32),
            jax.ShapeDtypeStruct((N, F), jnp.float32),
        ),
    )(t, st, g, be, fcw_row)


def kernel(x, edge_index, W0, b0, g0, be0, W1, b1, g1, be1, W2, b2, g2, be2,
           W3, b3, g3, be3, W4, b4, g4, be4, fcW, fcb):
    e3 = edge_index.reshape(2, E // K, K)
    Ws = [W0, W1, W2, W3, W4]
    bs = [b0, b1, b2, b3, b4]
    gs = [g0, g1, g2, g3, g4]
    bes = [be0, be1, be2, be3, be4]

    degp = _deg(e3, jnp.ones((K, 8), jnp.float32), jnp.zeros((NP, 8), jnp.float32))
    y, dinv = _first_tc(x, W0, degp)

    for i in range(5):
        NFC, _, FC = y.shape
        F = NFC * FC
        z = _spmm(y, e3, jnp.zeros((NP, FC), jnp.float32), NFC, FC)
        t, st = _stats_tc(z, y, dinv, bs[i].reshape(1, F))
        if i < 4:
            F2 = Ws[i + 1].shape[1]
            NFC2 = 2 if F2 > 128 else 1
            y = _apply_tc(t, st, gs[i].reshape(1, F), bes[i].reshape(1, F), dinv,
                          Ws[i + 1], NFC2, F2 // NFC2)
        else:
            h5, h5w = _apply_last_tc(t, st, gs[i].reshape(1, F), bes[i].reshape(1, F),
                                     fcW[:, 0].reshape(1, F))

    e3h = edge_index.reshape(2, E // KH, KH)
    fcb16 = jnp.broadcast_to(fcb.reshape(1), (16,))
    return _headdot(h5w, h5, e3h, fcb16).reshape(E, 1)


# speedup vs baseline: 21.2890x; 1.3216x over previous
import functools

import jax
import jax.numpy as jnp
from jax import lax
from jax.experimental import pallas as pl
from jax.experimental.pallas import tpu as pltpu
from jax.experimental.pallas import tpu_sc as plsc

N = 10000
NP = 10240
E = 320000
NC = 2
NS = 16
NW = NC * NS
K = 100
CH = E // (NW * K)
RP = NP // NS
RB = 1000
NB = N // RB

_MESH = plsc.VectorSubcoreMesh(core_axis_name="c", subcore_axis_name="s")
_SC_PARAMS = pltpu.CompilerParams(use_tc_tiling_on_sc=False)
_SC_VPARAMS = pltpu.CompilerParams(use_tc_tiling_on_sc=False, needs_layout_passes=False)


@functools.partial(jax.jit, static_argnums=(3, 4))
def _spmm(y, e3, zeros, NFC, FC):
    @functools.partial(
        pl.kernel,
        out_type=jax.ShapeDtypeStruct((NC, NFC, NP, FC), jnp.float32),
        mesh=_MESH,
        scratch_types=[
            pltpu.VMEM((CH, K), jnp.int32),
            pltpu.VMEM((CH, K), jnp.int32),
            pltpu.VMEM((K, FC), jnp.float32),
            pltpu.VMEM((K, FC), jnp.float32),
            pltpu.VMEM_SHARED((NP, FC), jnp.float32),
            pltpu.SemaphoreType.DMA,
            pltpu.SemaphoreType.DMA,
        ],
        compiler_params=_SC_PARAMS,
    )
    def spmm(y_hbm, e3_hbm, zeros_hbm, out_hbm, src_v, dst_v, rows0_v, rows1_v, zsh, sem0, sem1):
        c = lax.axis_index("c")
        s = lax.axis_index("s")
        w = s * NC + c
        pltpu.sync_copy(e3_hbm.at[0].at[pl.ds(w * CH, CH)], src_v)
        pltpu.sync_copy(e3_hbm.at[1].at[pl.ds(w * CH, CH)], dst_v)
        for f in range(NFC):
            yf = y_hbm.at[f]
            pltpu.sync_copy(zeros_hbm.at[pl.ds(s * RP, RP)], zsh.at[pl.ds(s * RP, RP)])
            pltpu.async_copy(yf.at[src_v.at[0]], rows0_v, sem0)
            pltpu.async_copy(yf.at[src_v.at[1]], rows1_v, sem1)
            plsc.subcore_barrier()

            def step(jj, carry):
                j0 = jj * 2
                pltpu.make_async_copy(yf.at[src_v.at[j0]], rows0_v, sem0).wait()
                pltpu.sync_copy(rows0_v, zsh.at[dst_v.at[j0]], add=True)
                pltpu.async_copy(yf.at[src_v.at[jnp.minimum(j0 + 2, CH - 1)]], rows0_v, sem0)
                pltpu.make_async_copy(yf.at[src_v.at[j0 + 1]], rows1_v, sem1).wait()
                pltpu.sync_copy(rows1_v, zsh.at[dst_v.at[j0 + 1]], add=True)
                pltpu.async_copy(yf.at[src_v.at[jnp.minimum(j0 + 3, CH - 1)]], rows1_v, sem1)
                return carry

            lax.fori_loop(0, CH // 2, step, 0)
            pltpu.make_async_copy(yf.at[src_v.at[CH - 1]], rows0_v, sem0).wait()
            pltpu.make_async_copy(yf.at[src_v.at[CH - 1]], rows1_v, sem1).wait()
            plsc.subcore_barrier()
            pltpu.sync_copy(zsh.at[pl.ds(s * RP, RP)],
                            out_hbm.at[c].at[f].at[pl.ds(s * RP, RP)])

    return spmm(y, e3, zeros)


@jax.jit
def _deg(e3, ones, zeros):
    @functools.partial(
        pl.kernel,
        out_type=jax.ShapeDtypeStruct((NC, NP, 8), jnp.float32),
        mesh=_MESH,
        scratch_types=[
            pltpu.VMEM((CH, K), jnp.int32),
            pltpu.VMEM((K, 8), jnp.float32),
            pltpu.VMEM_SHARED((NP, 8), jnp.float32),
        ],
        compiler_params=_SC_PARAMS,
    )
    def deg(e3_hbm, ones_hbm, zeros_hbm, out_hbm, dst_v, ones_v, zsh):
        c = lax.axis_index("c")
        s = lax.axis_index("s")
        w = s * NC + c
        pltpu.sync_copy(e3_hbm.at[1].at[pl.ds(w * CH, CH)], dst_v)
        pltpu.sync_copy(ones_hbm, ones_v)
        pltpu.sync_copy(zeros_hbm.at[pl.ds(s * RP, RP)], zsh.at[pl.ds(s * RP, RP)])
        plsc.subcore_barrier()

        def step(j, carry):
            pltpu.sync_copy(ones_v, zsh.at[dst_v.at[j]], add=True)
            return carry

        lax.fori_loop(0, CH, step, 0)
        plsc.subcore_barrier()
        pltpu.sync_copy(zsh.at[pl.ds(s * RP, RP)], out_hbm.at[c].at[pl.ds(s * RP, RP)])

    return deg(e3, ones, zeros)


KH = 80
CHH = E // (NW * KH)
EPW = E // NW


@jax.jit
def _headdot(hw, h, e3h, fcb16):
    HD = 32

    @functools.partial(
        pl.kernel,
        out_type=jax.ShapeDtypeStruct((E,), jnp.float32),
        mesh=_MESH,
        scratch_types=[
            pltpu.VMEM((CHH, KH), jnp.int32),
            pltpu.VMEM((CHH, KH), jnp.int32),
            pltpu.VMEM((KH, HD), jnp.float32),
            pltpu.VMEM((KH, HD), jnp.float32),
            pltpu.VMEM((KH, HD), jnp.float32),
            pltpu.VMEM((KH, HD), jnp.float32),
            pltpu.VMEM((16,), jnp.float32),
            pltpu.VMEM((EPW,), jnp.float32),
            pltpu.SemaphoreType.DMA,
            pltpu.SemaphoreType.DMA,
        ],
        compiler_params=_SC_VPARAMS,
    )
    def hd(hw_hbm, h_hbm, e3_hbm, fcb_hbm, out_hbm, src_v, dst_v,
           ra0_v, rb0_v, ra1_v, rb1_v, fcb_v, out_v, s0, s1):
        c = lax.axis_index("c")
        s = lax.axis_index("s")
        w = s * NC + c
        pltpu.sync_copy(e3_hbm.at[0].at[pl.ds(w * CHH, CHH)], src_v)
        pltpu.sync_copy(e3_hbm.at[1].at[pl.ds(w * CHH, CHH)], dst_v)
        pltpu.sync_copy(fcb_hbm, fcb_v)
        pltpu.async_copy(hw_hbm.at[src_v.at[0]], ra0_v, s0)
        pltpu.async_copy(h_hbm.at[dst_v.at[0]], rb0_v, s0)
        pltpu.async_copy(hw_hbm.at[src_v.at[1]], ra1_v, s1)
        pltpu.async_copy(h_hbm.at[dst_v.at[1]], rb1_v, s1)
        lanes = jnp.arange(16, dtype=jnp.int32)

        def process(j, ra_v, rb_v, sem, prefetch):
            pltpu.make_async_copy(hw_hbm.at[src_v.at[j]], ra_v, sem).wait()
            pltpu.make_async_copy(h_hbm.at[dst_v.at[j]], rb_v, sem).wait()
            for g in range(KH // 16):
                rows = lanes + (g * 16)
                accs = [fcb_v[...], None, None, None]
                for k in range(HD):
                    col = jnp.bitwise_and(lanes + k, HD - 1)
                    p = plsc.load_gather(ra_v, [rows, col]) * plsc.load_gather(rb_v, [rows, col])
                    a = accs[k % 4]
                    accs[k % 4] = p if a is None else a + p
                acc = (accs[0] + accs[1]) + (accs[2] + accs[3])
                out_v[pl.ds(j * KH + g * 16, 16)] = 1.0 / (1.0 + jnp.exp(-acc))
            if prefetch:
                jp = jnp.minimum(j + 2, CHH - 1)
                pltpu.async_copy(hw_hbm.at[src_v.at[jp]], ra_v, sem)
                pltpu.async_copy(h_hbm.at[dst_v.at[jp]], rb_v, sem)

        def step(jj, carry):
            process(jj * 2, ra0_v, rb0_v, s0, True)
            process(jj * 2 + 1, ra1_v, rb1_v, s1, True)
            return carry

        lax.fori_loop(0, CHH // 2, step, 0)
        process(CHH - 1, ra0_v, rb0_v, s0, False)
        pltpu.make_async_copy(hw_hbm.at[src_v.at[0]], ra1_v, s1).wait()
        pltpu.make_async_copy(h_hbm.at[dst_v.at[0]], rb1_v, s1).wait()
        pltpu.sync_copy(out_v, out_hbm.at[pl.ds(w * EPW, EPW)])

    return hd(hw, h, e3h, fcb16)


def _first_tc(x, W0, degp):
    F = W0.shape[1]

    def body(x_ref, w_ref, dp_ref, y_ref, dinv_ref):
        deg = dp_ref[0, :, 0:1] + dp_ref[1, :, 0:1] + 1.0
        dinv = lax.rsqrt(deg)
        dinv_ref[...] = dinv
        y_ref[0] = dinv * jnp.dot(x_ref[...], w_ref[...], preferred_element_type=jnp.float32)

    return pl.pallas_call(
        body,
        grid=(NB,),
        in_specs=[
            pl.BlockSpec((RB, x.shape[1]), lambda b: (b, 0)),
            pl.BlockSpec((x.shape[1], F), lambda b: (0, 0)),
            pl.BlockSpec((2, RB, 8), lambda b: (0, b, 0)),
        ],
        out_specs=(
            pl.BlockSpec((1, RB, F), lambda b: (0, b, 0)),
            pl.BlockSpec((RB, 1), lambda b: (b, 0)),
        ),
        out_shape=(
            jax.ShapeDtypeStruct((1, N, F), jnp.float32),
            jax.ShapeDtypeStruct((N, 1), jnp.float32),
        ),
    )(x, W0, degp)


def _stats_tc(z, y, dinv, b):
    NFC, _, FC = y.shape
    F = NFC * FC

    def body(z_ref, y_ref, dinv_ref, b_ref, t_ref, st_ref):
        i = pl.program_id(0)
        parts = [z_ref[0, f] + z_ref[1, f] + y_ref[f] for f in range(NFC)]
        zy = jnp.concatenate(parts, axis=1) if NFC > 1 else parts[0]
        t = dinv_ref[...] * zy + b_ref[...]
        t_ref[...] = t

        @pl.when(i == 0)
        def _():
            st_ref[...] = jnp.zeros_like(st_ref)

        st_ref[0:1, :] += jnp.sum(t, axis=0, keepdims=True)
        st_ref[1:2, :] += jnp.sum(t * t, axis=0, keepdims=True)

    return pl.pallas_call(
        body,
        grid=(NB,),
        in_specs=[
            pl.BlockSpec((2, NFC, RB, FC), lambda b: (0, 0, b, 0)),
            pl.BlockSpec((NFC, RB, FC), lambda b: (0, b, 0)),
            pl.BlockSpec((RB, 1), lambda b: (b, 0)),
            pl.BlockSpec((1, F), lambda b: (0, 0)),
        ],
        out_specs=(
            pl.BlockSpec((RB, F), lambda b: (b, 0)),
            pl.BlockSpec((2, F), lambda b: (0, 0)),
        ),
        out_shape=(
            jax.ShapeDtypeStruct((N, F), jnp.float32),
            jax.ShapeDtypeStruct((2, F), jnp.float32),
        ),
    )(z, y, dinv, b)


def _bn_relu(t_ref, st_ref, g_ref, be_ref):
    mu = st_ref[0:1, :] * (1.0 / N)
    var = st_ref[1:2, :] * (1.0 / N) - mu * mu
    inv = lax.rsqrt(var + 1e-5)
    return jnp.maximum(g_ref[...] * (t_ref[...] - mu) * inv + be_ref[...], 0.0)


def _apply_tc(t, st, g, be, dinv, W, NFC2, FC2):
    F = t.shape[1]

    def body(t_ref, st_ref, g_ref, be_ref, dinv_ref, w_ref, o_ref):
        h = _bn_relu(t_ref, st_ref, g_ref, be_ref)
        yp = dinv_ref[...] * jnp.dot(h, w_ref[...], preferred_element_type=jnp.float32)
        for f in range(NFC2):
            o_ref[f] = yp[:, f * FC2:(f + 1) * FC2]

    return pl.pallas_call(
        body,
        grid=(NB,),
        in_specs=[
            pl.BlockSpec((RB, F), lambda b: (b, 0)),
            pl.BlockSpec((2, F), lambda b: (0, 0)),
            pl.BlockSpec((1, F), lambda b: (0, 0)),
            pl.BlockSpec((1, F), lambda b: (0, 0)),
            pl.BlockSpec((RB, 1), lambda b: (b, 0)),
            pl.BlockSpec((F, NFC2 * FC2), lambda b: (0, 0)),
        ],
        out_specs=pl.BlockSpec((NFC2, RB, FC2), lambda b: (0, b, 0)),
        out_shape=jax.ShapeDtypeStruct((NFC2, N, FC2), jnp.float32),
    )(t, st, g, be, dinv, W)


def _apply_last_tc(t, st, g, be, fcw_row):
    F = t.shape[1]

    def body(t_ref, st_ref, g_ref, be_ref, fw_ref, h_ref, hw_ref):
        h = _bn_relu(t_ref, st_ref, g_ref, be_ref)
        h_ref[...] = h
        hw_ref[...] = h * fw_ref[...]

    return pl.pallas_call(
        body,
        grid=(NB,),
        in_specs=[
            pl.BlockSpec((RB, F), lambda b: (b, 0)),
            pl.BlockSpec((2, F), lambda b: (0, 0)),
            pl.BlockSpec((1, F), lambda b: (0, 0)),
            pl.BlockSpec((1, F), lambda b: (0, 0)),
            pl.BlockSpec((1, F), lambda b: (0, 0)),
        ],
        out_specs=(
            pl.BlockSpec((RB, F), lambda b: (b, 0)),
            pl.BlockSpec((RB, F), lambda b: (b, 0)),
        ),
        out_shape=(
            jax.ShapeDtypeStruct((N, F), jnp.float32),
            jax.ShapeDtypeStruct((N, F), jnp.float32),
        ),
    )(t, st, g, be, fcw_row)


def kernel(x, edge_index, W0, b0, g0, be0, W1, b1, g1, be1, W2, b2, g2, be2,
           W3, b3, g3, be3, W4, b4, g4, be4, fcW, fcb):
    e3 = edge_index.reshape(2, E // K, K)
    Ws = [W0, W1, W2, W3, W4]
    bs = [b0, b1, b2, b3, b4]
    gs = [g0, g1, g2, g3, g4]
    bes = [be0, be1, be2, be3, be4]

    degp = _deg(e3, jnp.ones((K, 8), jnp.float32), jnp.zeros((NP, 8), jnp.float32))
    y, dinv = _first_tc(x, W0, degp)

    for i in range(5):
        NFC, _, FC = y.shape
        F = NFC * FC
        z = _spmm(y, e3, jnp.zeros((NP, FC), jnp.float32), NFC, FC)
        t, st = _stats_tc(z, y, dinv, bs[i].reshape(1, F))
        if i < 4:
            F2 = Ws[i + 1].shape[1]
            NFC2 = 2 if F2 > 128 else 1
            y = _apply_tc(t, st, gs[i].reshape(1, F), bes[i].reshape(1, F), dinv,
                          Ws[i + 1], NFC2, F2 // NFC2)
        else:
            h5, h5w = _apply_last_tc(t, st, gs[i].reshape(1, F), bes[i].reshape(1, F),
                                     fcW[:, 0].reshape(1, F))

    e3h = edge_index.reshape(2, E // KH, KH)
    fcb16 = jnp.broadcast_to(fcb.reshape(1), (16,))
    return _headdot(h5w, h5, e3h, fcb16).reshape(E, 1)
